# Initial kernel scaffold; baseline (speedup 1.0000x reference)
import jax
import jax.numpy as jnp
from jax.experimental import pallas as pl


def kernel(sgt_trans_mat, use_gumbel_noise, is_training):
    raise NotImplementedError("write your pallas kernel here")


# speedup vs baseline: 7.6473x; 7.6473x over previous
import functools

import jax
import jax.numpy as jnp
from jax import lax
from jax.experimental import pallas as pl
from jax.experimental.pallas import tpu as pltpu
from jax.experimental.pallas import tpu_sc as plsc

J = 133
HW = 262144
N = HW * J
NC = 2
NS = 16
L = 16
NW = NC * NS
ROWS_PER_W = HW // NW
CHUNK_ROWS = 128
CHUNK = CHUNK_ROWS * J
CHUNKS_PER_W = ROWS_PER_W // CHUNK_ROWS
PAIRS = CHUNKS_PER_W // 2
GROUPS = CHUNK_ROWS // L
HEAD_WORDS = 17696
KV = 9

_mesh = plsc.VectorSubcoreMesh(core_axis_name="c", subcore_axis_name="s")


@functools.partial(
    pl.kernel,
    mesh=_mesh,
    compiler_params=pltpu.CompilerParams(needs_layout_passes=False),
    out_type=jax.ShapeDtypeStruct((N,), jnp.float32),
    scratch_types=[
        pltpu.VMEM((2 * CHUNK,), jnp.float32),
        pltpu.VMEM((2 * CHUNK,), jnp.float32),
        pltpu.VMEM((HEAD_WORDS + L,), jnp.float32),
        pltpu.VMEM((160,), jnp.int32),
        pltpu.VMEM((KV * L,), jnp.float32),
        pltpu.VMEM((2 * CHUNK_ROWS,), jnp.int32),
        pltpu.SemaphoreType.DMA,
        pltpu.SemaphoreType.DMA,
        pltpu.SemaphoreType.DMA,
        pltpu.SemaphoreType.DMA,
        pltpu.SemaphoreType.DMA,
    ],
)
def _sc_onehot(x_hbm, out_hbm, in_buf, out_buf, head_buf, sel_ref, avail_ref,
               pos_ref, sem_in0, sem_in1, sem_out0, sem_out1, sem_head):
    lanes = lax.iota(jnp.int32, L)
    wid = lax.axis_index("s") * NC + lax.axis_index("c")
    base_w = wid * ROWS_PER_W * J

    zeros_f = jnp.zeros((L,), jnp.float32)
    ones_f = jnp.ones((L,), jnp.float32)
    neg_inf = jnp.full((L,), -jnp.inf, jnp.float32)
    lane0 = lanes == 0

    def in_copy(c_idx, b):
        sem = sem_in0 if b == 0 else sem_in1
        return pltpu.make_async_copy(
            x_hbm.at[pl.ds(base_w + c_idx * CHUNK, CHUNK)],
            in_buf.at[pl.ds(b * CHUNK, CHUNK)], sem)

    def out_copy(c_idx, b):
        sem = sem_out0 if b == 0 else sem_out1
        return pltpu.make_async_copy(
            out_buf.at[pl.ds(b * CHUNK, CHUNK)],
            out_hbm.at[pl.ds(base_w + c_idx * CHUNK, CHUNK)], sem)

    in_copy(0, 0).start()
    in_copy(1, 1).start()

    def _zero(i, carry):
        for k in range(8):
            out_buf[pl.ds(i * 128 + k * 16, 16)] = zeros_f
        return carry
    lax.fori_loop(0, (2 * CHUNK) // 128, _zero, 0)
    for g in range(2 * GROUPS):
        pos_ref[pl.ds(g * 16, 16)] = jnp.zeros((L,), jnp.int32)

    @pl.when(wid == 0)
    def _greedy():
        hd = pltpu.make_async_copy(
            x_hbm.at[pl.ds(0, HEAD_WORDS)],
            head_buf.at[pl.ds(0, HEAD_WORDS)], sem_head)
        hd.start()
        hd.wait()
        for k in range(KV):
            avail_ref[pl.ds(k * 16, 16)] = ones_f
        colmask = [(k * 16 + lanes) < J for k in range(KV)]
        colvec = [k * 16 + lanes for k in range(KV)]

        def gbody(i, carry):
            roff = i * J
            bestv = neg_inf
            besti = jnp.zeros((L,), jnp.int32)
            vs = []
            for k in range(KV):
                v = head_buf[pl.ds(roff + k * 16, 16)]
                vs.append(v)
                av = avail_ref[pl.ds(k * 16, 16)]
                m = jnp.where((av > 0.0) & colmask[k], v, neg_inf)
                take = m > bestv
                bestv = jnp.where(take, m, bestv)
                besti = jnp.where(take, colvec[k], besti)
            mx = bestv[0]
            c = besti[0]
            for l in range(1, L):
                v_l = bestv[l]
                i_l = besti[l]
                better = (v_l > mx) | ((v_l == mx) & (i_l < c))
                mx = jnp.where(better, v_l, mx)
                c = jnp.where(better, i_l, c)
            vc = plsc.load_gather(
                head_buf, [jnp.full((L,), roff, jnp.int32) + c])
            rank_v = jnp.zeros((L,), jnp.int32)
            for k in range(KV):
                gtm = (vs[k] > vc) & colmask[k]
                eqm = (vs[k] == vc) & (colvec[k] < c) & colmask[k]
                rank_v = rank_v + gtm.astype(jnp.int32) + eqm.astype(jnp.int32)
            rank = rank_v[0]
            for l in range(1, L):
                rank = rank + rank_v[l]
            found = rank <= i
            selv = jnp.where(found, c, 0)
            plsc.store_scatter(sel_ref, [jnp.full((L,), i, jnp.int32)],
                               jnp.full((L,), selv, jnp.int32), mask=lane0)
            plsc.store_scatter(avail_ref, [jnp.full((L,), 0, jnp.int32) + c],
                               zeros_f, mask=lane0 & found)
            return carry
        lax.fori_loop(0, J, gbody, 0)

    def pbody(p, carry):
        for b in range(2):
            c_idx = 2 * p + b
            in_copy(c_idx, b).wait()

            @pl.when(p >= 1)
            def _drain_out():
                out_copy(c_idx - 2, b).wait()

            for g in range(GROUPS):
                pv = pos_ref[pl.ds(b * CHUNK_ROWS + g * 16, 16)]
                plsc.store_scatter(out_buf, [pv], zeros_f)

            def gloop(g, carry2):
                base = b * CHUNK + (g * 16 + lanes) * J
                maxv = neg_inf
                maxi = jnp.zeros((L,), jnp.int32)
                for j in range(J):
                    v = plsc.load_gather(in_buf, [base + j])
                    take = v > maxv
                    maxv = jnp.where(take, v, maxv)
                    maxi = jnp.where(take, jnp.full((L,), j, jnp.int32), maxi)
                grow0 = wid * ROWS_PER_W + c_idx * CHUNK_ROWS + g * 16
                soff = jnp.minimum(grow0, 144)
                selv = sel_ref[pl.ds(soff, 16)]
                col = jnp.where(grow0 + lanes < J, selv, maxi)
                pos = base + col
                plsc.store_scatter(out_buf, [pos], ones_f)
                pos_ref[pl.ds(b * CHUNK_ROWS + g * 16, 16)] = pos
                return carry2
            lax.fori_loop(0, GROUPS, gloop, 0)

            out_copy(c_idx, b).start()

            @pl.when(p <= PAIRS - 2)
            def _prefetch():
                in_copy(c_idx + 2, b).start()
        return carry
    lax.fori_loop(0, PAIRS, pbody, 0)

    out_copy(CHUNKS_PER_W - 2, 0).wait()
    out_copy(CHUNKS_PER_W - 1, 1).wait()


def kernel(sgt_trans_mat, use_gumbel_noise, is_training):
    del use_gumbel_noise, is_training
    x = sgt_trans_mat.reshape(-1)
    out = _sc_onehot(x)
    return out.reshape(sgt_trans_mat.shape)


# speedup vs baseline: 10.0171x; 1.3099x over previous
import functools

import jax
import jax.numpy as jnp
from jax import lax
from jax.experimental import pallas as pl
from jax.experimental.pallas import tpu as pltpu
from jax.experimental.pallas import tpu_sc as plsc

J = 133
HW = 262144
NC = 2
NS = 16
L = 16
NW = NC * NS
ROWS_PER_W = HW // NW
CHUNK_ROWS = 64
CHUNKS_PER_W = ROWS_PER_W // CHUNK_ROWS
PAIRS = CHUNKS_PER_W // 2
GROUPS = CHUNK_ROWS // L
HEAD_ROWS = 144
KV = 9
JSPLIT = 67

_mesh = plsc.VectorSubcoreMesh(core_axis_name="c", subcore_axis_name="s")


@functools.partial(
    pl.kernel,
    mesh=_mesh,
    compiler_params=pltpu.CompilerParams(needs_layout_passes=False),
    out_type=jax.ShapeDtypeStruct((HW, J), jnp.float32),
    scratch_types=[
        pltpu.VMEM((2, CHUNK_ROWS, J), jnp.float32),
        pltpu.VMEM((2, CHUNK_ROWS, J), jnp.float32),
        pltpu.VMEM((HEAD_ROWS, J), jnp.float32),
        pltpu.VMEM((160,), jnp.int32),
        pltpu.VMEM((KV * L,), jnp.float32),
        pltpu.VMEM((2 * CHUNK_ROWS,), jnp.int32),
        pltpu.SemaphoreType.DMA,
        pltpu.SemaphoreType.DMA,
        pltpu.SemaphoreType.DMA,
        pltpu.SemaphoreType.DMA,
        pltpu.SemaphoreType.DMA,
    ],
)
def _sc_onehot(x_hbm, out_hbm, in_buf, out_buf, head_buf, sel_ref, avail_ref,
               pos_ref, sem_in0, sem_in1, sem_out0, sem_out1, sem_head):
    lanes = lax.iota(jnp.int32, L)
    wid = lax.axis_index("s") * NC + lax.axis_index("c")
    row0_w = wid * ROWS_PER_W

    zeros_f = jnp.zeros((L,), jnp.float32)
    ones_f = jnp.ones((L,), jnp.float32)
    neg_inf = jnp.full((L,), -jnp.inf, jnp.float32)
    lane0 = lanes == 0
    colmask = [(k * 16 + lanes) < J for k in range(KV)]
    colvec = [k * 16 + lanes for k in range(KV)]

    def in_copy(c_idx, b):
        sem = sem_in0 if b == 0 else sem_in1
        return pltpu.make_async_copy(
            x_hbm.at[pl.ds(row0_w + c_idx * CHUNK_ROWS, CHUNK_ROWS), :],
            in_buf.at[b], sem)

    def out_copy(c_idx, b):
        sem = sem_out0 if b == 0 else sem_out1
        return pltpu.make_async_copy(
            out_buf.at[b],
            out_hbm.at[pl.ds(row0_w + c_idx * CHUNK_ROWS, CHUNK_ROWS), :], sem)

    in_copy(0, 0).start()
    in_copy(1, 1).start()

    tailc = 128 + lanes
    tailmask = lanes < (J - 128)
    for b in range(2):
        def _zrow(r, carry, b=b):
            for k in range(8):
                out_buf[b, r, pl.ds(k * 16, 16)] = zeros_f
            plsc.store_scatter(
                out_buf.at[b], [jnp.full((L,), 0, jnp.int32) + r, tailc],
                zeros_f, mask=tailmask)
            return carry
        lax.fori_loop(0, CHUNK_ROWS, _zrow, 0)
    for g in range(2 * GROUPS):
        pos_ref[pl.ds(g * 16, 16)] = jnp.zeros((L,), jnp.int32)

    @pl.when(wid == 0)
    def _greedy():
        hd = pltpu.make_async_copy(
            x_hbm.at[pl.ds(0, HEAD_ROWS), :], head_buf, sem_head)
        hd.start()
        hd.wait()
        for k in range(KV):
            avail_ref[pl.ds(k * 16, 16)] = ones_f

        def gbody(i, carry):
            ivec = jnp.full((L,), 0, jnp.int32) + i
            bestv = neg_inf
            besti = jnp.zeros((L,), jnp.int32)
            vs = []
            for k in range(KV):
                v = plsc.load_gather(head_buf, [ivec, colvec[k]],
                                     mask=colmask[k])
                vs.append(v)
                av = avail_ref[pl.ds(k * 16, 16)]
                m = jnp.where((av > 0.0) & colmask[k], v, neg_inf)
                take = m > bestv
                bestv = jnp.where(take, m, bestv)
                besti = jnp.where(take, colvec[k], besti)
            mx = bestv[0]
            c = besti[0]
            for l in range(1, L):
                v_l = bestv[l]
                i_l = besti[l]
                better = (v_l > mx) | ((v_l == mx) & (i_l < c))
                mx = jnp.where(better, v_l, mx)
                c = jnp.where(better, i_l, c)
            vc = plsc.load_gather(head_buf,
                                  [ivec, jnp.full((L,), 0, jnp.int32) + c])
            rank_v = jnp.zeros((L,), jnp.int32)
            for k in range(KV):
                gtm = (vs[k] > vc) & colmask[k]
                eqm = (vs[k] == vc) & (colvec[k] < c) & colmask[k]
                rank_v = rank_v + gtm.astype(jnp.int32) + eqm.astype(jnp.int32)
            rank = rank_v[0]
            for l in range(1, L):
                rank = rank + rank_v[l]
            found = rank <= i
            selv = jnp.where(found, c, 0)
            plsc.store_scatter(sel_ref, [ivec],
                               jnp.full((L,), selv, jnp.int32), mask=lane0)
            plsc.store_scatter(avail_ref, [jnp.full((L,), 0, jnp.int32) + c],
                               zeros_f, mask=lane0 & found)
            return carry
        lax.fori_loop(0, J, gbody, 0)

    def pbody(p, carry):
        for b in range(2):
            c_idx = 2 * p + b
            in_copy(c_idx, b).wait()

            @pl.when(p >= 1)
            def _drain_out():
                out_copy(c_idx - 2, b).wait()

            for g in range(GROUPS):
                rowv = g * 16 + lanes
                pc = pos_ref[pl.ds(b * CHUNK_ROWS + g * 16, 16)]
                plsc.store_scatter(out_buf.at[b], [rowv, pc], zeros_f)

            def gloop(g, carry2, b=b, c_idx=c_idx):
                rowv = g * 16 + lanes
                maxv0 = neg_inf
                maxi0 = jnp.zeros((L,), jnp.int32)
                maxv1 = neg_inf
                maxi1 = jnp.zeros((L,), jnp.int32)
                for j in range(JSPLIT):
                    v = plsc.load_gather(in_buf.at[b],
                                         [rowv, jnp.full((L,), j, jnp.int32)])
                    take = v > maxv0
                    maxv0 = jnp.where(take, v, maxv0)
                    maxi0 = jnp.where(take, jnp.full((L,), j, jnp.int32),
                                      maxi0)
                for j in range(JSPLIT, J):
                    v = plsc.load_gather(in_buf.at[b],
                                         [rowv, jnp.full((L,), j, jnp.int32)])
                    take = v > maxv1
                    maxv1 = jnp.where(take, v, maxv1)
                    maxi1 = jnp.where(take, jnp.full((L,), j, jnp.int32),
                                      maxi1)
                hi = maxv1 > maxv0
                maxi = jnp.where(hi, maxi1, maxi0)
                grow0 = row0_w + c_idx * CHUNK_ROWS + g * 16
                soff = jnp.minimum(grow0, 144)
                selv = sel_ref[pl.ds(soff, 16)]
                col = jnp.where(grow0 + lanes < J, selv, maxi)
                plsc.store_scatter(out_buf.at[b], [rowv, col], ones_f)
                pos_ref[pl.ds(b * CHUNK_ROWS + g * 16, 16)] = col
                return carry2
            lax.fori_loop(0, GROUPS, gloop, 0)

            out_copy(c_idx, b).start()

            @pl.when(p <= PAIRS - 2)
            def _prefetch():
                in_copy(c_idx + 2, b).start()
        return carry
    lax.fori_loop(0, PAIRS, pbody, 0)

    out_copy(CHUNKS_PER_W - 2, 0).wait()
    out_copy(CHUNKS_PER_W - 1, 1).wait()


def kernel(sgt_trans_mat, use_gumbel_noise, is_training):
    del use_gumbel_noise, is_training
    return _sc_onehot(sgt_trans_mat)
